# baseline (speedup 1.0000x reference)
import jax
import jax.numpy as jnp
from jax import lax
from jax.experimental import pallas as pl
from jax.experimental.pallas import tpu as pltpu
from jax.experimental.pallas import tpu_sc as plsc

EMBED = 16
REGS = 1e-05
B = 16384
NROWS = 1000000
PACK = 8
WIDE = EMBED * PACK
NPROWS = NROWS // PACK
NC, NS, L = 2, 16, 16
NW = NC * NS
BPW = B // NW
CHUNK = 128
NCHUNK = BPW // CHUNK
BLKC = CHUNK // L


def _softplus_neg_abs(l):
    t = jnp.exp(-jnp.abs(l))
    s = t / (t + 2.0)
    s2 = s * s
    return 2.0 * s * (1.0 + s2 * (1.0 / 3.0 + s2 * (0.2 + s2 * (1.0 / 7.0))))


def _disc_kernel(user, pos, neg, uemb, iemb, bias, out,
                 idx_u, idx_p, idx_n, pr_u, pr_p, pr_n,
                 u_buf, p_buf, n_buf, b_p, b_n, stage, sem):
    wid = lax.axis_index("s") * NC + lax.axis_index("c")
    base = wid * BPW

    h0 = pltpu.async_copy(user.at[pl.ds(base, BPW)], idx_u, sem)
    h1 = pltpu.async_copy(pos.at[pl.ds(base, BPW)], idx_p, sem)
    h2 = pltpu.async_copy(neg.at[pl.ds(base, BPW)], idx_n, sem)
    h0.wait()
    h1.wait()
    h2.wait()

    for raw, pr in ((idx_u, pr_u), (idx_p, pr_p), (idx_n, pr_n)):
        for j in range(BPW // L):
            sl = pl.ds(j * L, L)
            pr[sl] = raw[sl] >> 3

    hb = []
    for c in range(NCHUNK):
        sl = pl.ds(c * CHUNK, CHUNK)
        hb.append(pltpu.async_copy(bias.at[idx_p.at[sl]], b_p.at[sl], sem))
        hb.append(pltpu.async_copy(bias.at[idx_n.at[sl]], b_n.at[sl], sem))

    lanes = lax.iota(jnp.int32, L)
    zero = jnp.zeros((L,), jnp.float32)
    carry = (zero, zero, zero, zero)

    for c in range(NCHUNK):
        sl = pl.ds(c * CHUNK, CHUNK)
        g0 = pltpu.async_copy(uemb.at[pr_u.at[sl]], u_buf, sem)
        g1 = pltpu.async_copy(iemb.at[pr_p.at[sl]], p_buf, sem)
        g2 = pltpu.async_copy(iemb.at[pr_n.at[sl]], n_buf, sem)
        g0.wait()
        g1.wait()
        g2.wait()

        def block(j, carry, c=c):
            cls_acc, su, sp_, sn = carry
            r0 = c * CHUNK + j * L
            rows = j * L + lanes
            cu = (idx_u[pl.ds(r0, L)] & 7) * EMBED
            cp = (idx_p[pl.ds(r0, L)] & 7) * EMBED
            cn = (idx_n[pl.ds(r0, L)] & 7) * EMBED
            dp = zero
            dn = zero
            for d in range(EMBED):
                u = plsc.load_gather(u_buf, [rows, cu + d])
                p = plsc.load_gather(p_buf, [rows, cp + d])
                n = plsc.load_gather(n_buf, [rows, cn + d])
                dp = dp + u * p
                dn = dn + u * n
                su = su + u * u
                sp_ = sp_ + p * p
                sn = sn + n * n
            lp = dp + b_p[pl.ds(r0, L)]
            ln = dn + b_n[pl.ds(r0, L)]
            pos_t = jnp.maximum(lp, 0.0) - lp + _softplus_neg_abs(lp)
            neg_t = jnp.maximum(ln, 0.0) + _softplus_neg_abs(ln)
            return (cls_acc + pos_t + neg_t, su, sp_, sn)

        carry = lax.fori_loop(0, BLKC, block, carry, unroll=2)

    for h in hb:
        h.wait()
    cls_acc, su, sp_, sn = carry

    cls_s = jnp.sum(cls_acc)
    sq_s = jnp.sum(2.0 * su + sp_ + sn)
    stage[...] = jnp.where(lanes == 0, cls_s,
                           jnp.where(lanes == 1, sq_s, 0.0))
    pltpu.sync_copy(stage, out.at[wid])


@jax.jit
def kernel(user, pos, neg, user_embedding, item_embedding, bias):
    mesh = plsc.VectorSubcoreMesh(
        core_axis_name="c", subcore_axis_name="s",
        num_cores=NC, num_subcores=NS)
    k = pl.kernel(
        _disc_kernel,
        out_type=jax.ShapeDtypeStruct((NW, L), jnp.float32),
        mesh=mesh,
        compiler_params=pltpu.CompilerParams(
            needs_layout_passes=False, use_tc_tiling_on_sc=False),
        scratch_types=[
            pltpu.VMEM((BPW,), jnp.int32),
            pltpu.VMEM((BPW,), jnp.int32),
            pltpu.VMEM((BPW,), jnp.int32),
            pltpu.VMEM((BPW,), jnp.int32),
            pltpu.VMEM((BPW,), jnp.int32),
            pltpu.VMEM((BPW,), jnp.int32),
            pltpu.VMEM((CHUNK, WIDE), jnp.float32),
            pltpu.VMEM((CHUNK, WIDE), jnp.float32),
            pltpu.VMEM((CHUNK, WIDE), jnp.float32),
            pltpu.VMEM((BPW,), jnp.float32),
            pltpu.VMEM((BPW,), jnp.float32),
            pltpu.VMEM((L,), jnp.float32),
            pltpu.SemaphoreType.DMA,
        ],
    )
    part = k(user.astype(jnp.int32), pos.astype(jnp.int32),
             neg.astype(jnp.int32),
             user_embedding.reshape(NPROWS, WIDE),
             item_embedding.reshape(NPROWS, WIDE), bias)
    cls_loss = jnp.sum(part[:, 0]) / B
    reg_loss = jnp.float32(REGS * 0.5 / B) * jnp.sum(part[:, 1])
    return (cls_loss, reg_loss)
